# Initial kernel scaffold; baseline (speedup 1.0000x reference)
import jax
import jax.numpy as jnp
from jax.experimental import pallas as pl


def kernel(x, w_router, w_fc, w_proj):
    raise NotImplementedError("write your pallas kernel here")


# speedup vs baseline: 1.1230x; 1.1230x over previous
import functools

import jax
import jax.numpy as jnp
from jax.experimental import pallas as pl
from jax.experimental.pallas import tpu as pltpu

_B, _T, _C = 2, 4096, 1024
_H = 4 * _C
_N = _B * _T
_BT = 1024
_BH = 1024
_NT = _N // _BT
_NH = _H // _BH


def _dense_body(x_ref, wr_ref, wfc_ref, wp_ref, o_ref, h_ref):
    j = pl.program_id(1)

    @pl.when(j == 0)
    def _():
        xb = x_ref[...]
        ms = jnp.mean(jnp.square(xb), axis=-1, keepdims=True)
        h_ref[...] = xb * jax.lax.rsqrt(ms + 1e-6)
        o_ref[...] = jnp.zeros_like(o_ref)

    a = jax.lax.dot_general(h_ref[...], wfc_ref[...], (((1,), (0,)), ((), ())),
                            preferred_element_type=jnp.float32)
    a = jnp.maximum(a, 0.0)
    a = a * a
    o_ref[...] += jax.lax.dot_general(a, wp_ref[...], (((1,), (0,)), ((), ())),
                                      preferred_element_type=jnp.float32)

    @pl.when(j == _NH - 1)
    def _():
        xb = x_ref[...]
        logits = jax.lax.dot_general(xb, wr_ref[...], (((1,), (0,)), ((), ())),
                                     preferred_element_type=jnp.float32)
        mask = (logits > 0.0).astype(jnp.float32)
        o_ref[...] = xb + o_ref[...] * mask


@functools.partial(jax.jit, static_argnums=())
def _dense(x2d, w_router, w_fc, w_proj):
    return pl.pallas_call(
        _dense_body,
        grid=(_NT, _NH),
        in_specs=[
            pl.BlockSpec((_BT, _C), lambda i, j: (i, 0)),
            pl.BlockSpec((_C, 1), lambda i, j: (0, 0)),
            pl.BlockSpec((_C, _BH), lambda i, j: (0, j)),
            pl.BlockSpec((_BH, _C), lambda i, j: (j, 0)),
        ],
        out_specs=pl.BlockSpec((_BT, _C), lambda i, j: (i, 0)),
        out_shape=jax.ShapeDtypeStruct((_N, _C), jnp.float32),
        scratch_shapes=[pltpu.VMEM((_BT, _C), jnp.float32)],
        compiler_params=pltpu.CompilerParams(
            dimension_semantics=("arbitrary", "arbitrary"),
        ),
    )(x2d, w_router, w_fc, w_proj)


def kernel(x, w_router, w_fc, w_proj):
    x2d = x.reshape(_N, _C)
    out = _dense(x2d, w_router, w_fc, w_proj)
    return out.reshape(_B, _T, _C)
